# baseline (speedup 1.0000x reference)
import functools

import jax
import jax.numpy as jnp
from jax import lax
from jax.experimental import pallas as pl
from jax.experimental.pallas import tpu as pltpu
from jax.experimental.pallas import tpu_sc as plsc

N_NODES = 10000
N_EDGES = 320000
D = 128

NODES_PAD = 10112
TRASH_ROW = 10100
CHUNK = 128
NC = 2
NS = 16
NW = NC * NS
ROWS_PER_TILE = NODES_PAD // NS

CHUNKS_PER_TILE = 80
N_CHUNKS = NW * CHUNKS_PER_TILE
EDGES_PAD = N_CHUNKS * CHUNK
NBUF = 2


def _sc_segment_sum(x, src2, dst2, zrow, zdeg, ones):
    mesh = plsc.VectorSubcoreMesh(core_axis_name="c", subcore_axis_name="s")

    @functools.partial(
        pl.kernel,
        mesh=mesh,
        compiler_params=pltpu.CompilerParams(use_tc_tiling_on_sc=False),
        out_type=[
            jax.ShapeDtypeStruct((NC, NODES_PAD, D), jnp.float32),
            jax.ShapeDtypeStruct((NC, NODES_PAD, 16), jnp.float32),
        ],
        scratch_types=(
            [pltpu.VMEM((CHUNK,), jnp.int32) for _ in range(NBUF)]
            + [pltpu.VMEM((1, CHUNK), jnp.int32) for _ in range(NBUF)]
            + [pltpu.VMEM((CHUNK, D), jnp.float32) for _ in range(NBUF)]
            + [
                pltpu.VMEM((CHUNK, 16), jnp.float32),
                pltpu.VMEM_SHARED((NODES_PAD, D), jnp.float32),
                pltpu.VMEM_SHARED((NODES_PAD, 16), jnp.float32),
            ]
            + [pltpu.SemaphoreType.DMA for _ in range(3 * NBUF)]
        ),
    )
    def seg_kernel(x_hbm, src_hbm, dst_hbm, zrow_hbm, zdeg_hbm, ones_hbm,
                   pooled_hbm, deg_hbm, *refs):
        src_r = refs[0:NBUF]
        dst_r = refs[NBUF:2 * NBUF]
        buf_r = refs[2 * NBUF:3 * NBUF]
        ones_v = refs[3 * NBUF]
        accum_sh = refs[3 * NBUF + 1]
        deg_sh = refs[3 * NBUF + 2]
        gsem = refs[3 * NBUF + 3:4 * NBUF + 3]
        ssem = refs[4 * NBUF + 3:5 * NBUF + 3]
        dsem = refs[5 * NBUF + 3:6 * NBUF + 3]
        buf_a = buf_r[0]
        cid = lax.axis_index("c")
        sid = lax.axis_index("s")
        wid = sid * NC + cid

        r0 = sid * ROWS_PER_TILE
        pltpu.sync_copy(zrow_hbm, accum_sh.at[pl.ds(r0, ROWS_PER_TILE)])
        pltpu.sync_copy(zdeg_hbm, deg_sh.at[pl.ds(r0, ROWS_PER_TILE)])
        pltpu.sync_copy(ones_hbm, ones_v)
        e_base = wid * CHUNKS_PER_TILE * CHUNK
        plsc.subcore_barrier()

        def load_idx(k, i):
            e0 = e_base + k * CHUNK
            pltpu.sync_copy(src_hbm.at[pl.ds(e0, CHUNK)], src_r[i])
            pltpu.sync_copy(dst_hbm.at[pl.ds(e0, CHUNK)], dst_r[i].at[0])

        def gather(i):
            pltpu.make_async_copy(x_hbm.at[src_r[i]], buf_r[i], gsem[i]).start()

        def gwait(i):
            pltpu.make_async_copy(x_hbm.at[src_r[0]], buf_r[i], gsem[i]).wait()

        def scatter_start(i):
            pltpu.make_async_copy(
                buf_r[i], accum_sh.at[dst_r[i].at[0]], ssem[i]
            ).start(add=True)
            pltpu.make_async_copy(
                ones_v, deg_sh.at[dst_r[i].at[0]], dsem[i]
            ).start(add=True)

        def scatter_wait(i):
            pltpu.make_async_copy(
                buf_r[i], accum_sh.at[dst_r[i].at[0]], ssem[i]
            ).wait()
            pltpu.make_async_copy(
                ones_v, deg_sh.at[dst_r[i].at[0]], dsem[i]
            ).wait()

        for i in range(NBUF):
            load_idx(i, i)
            gather(i)

        def body(j, carry):
            k0 = NBUF * j
            for i in range(NBUF):
                gwait(i)
                scatter_start(i)

            @pl.when(j < CHUNKS_PER_TILE // NBUF - 1)
            def _():
                for i in range(NBUF):
                    scatter_wait(i)
                    load_idx(k0 + NBUF + i, i)
                    gather(i)

            @pl.when(j == CHUNKS_PER_TILE // NBUF - 1)
            def _():
                for i in range(NBUF):
                    scatter_wait(i)

            return carry

        lax.fori_loop(0, CHUNKS_PER_TILE // NBUF, body, 0)
        plsc.subcore_barrier()

        pltpu.sync_copy(accum_sh.at[pl.ds(r0, ROWS_PER_TILE)],
                        pooled_hbm.at[cid, pl.ds(r0, ROWS_PER_TILE)])
        pltpu.sync_copy(deg_sh.at[pl.ds(r0, ROWS_PER_TILE)],
                        deg_hbm.at[cid, pl.ds(r0, ROWS_PER_TILE)])

    return seg_kernel(x, src2, dst2, zrow, zdeg, ones)


def _tc_self_body(x, ws, out):
    out[...] = jnp.dot(x[...], ws[...], preferred_element_type=jnp.float32)


def _tc_combine_body(p0, p1, d0, d1, s, we, b, out):
    pooled = p0[...] + p1[...]
    e = jnp.dot(pooled, we[...], preferred_element_type=jnp.float32)
    denom = d0[:, 0:1] + d1[:, 0:1] + 1.0
    out[...] = jnp.maximum((e + s[...]) / denom + b[...], 0.0)


def kernel(x, edge_index, W_edge, W_self, b):
    src = edge_index[0]
    dst = edge_index[1]
    pad = EDGES_PAD - N_EDGES
    pad_idx = jnp.arange(pad, dtype=jnp.int32)
    src2 = jnp.concatenate([src, pad_idx % N_NODES])
    dst2 = jnp.concatenate([dst, N_NODES + pad_idx % (NODES_PAD - N_NODES)])
    zrow = jnp.zeros((ROWS_PER_TILE, D), jnp.float32)
    zdeg = jnp.zeros((ROWS_PER_TILE, 16), jnp.float32)
    ones = jnp.ones((CHUNK, 16), jnp.float32)

    pooled, deg = _sc_segment_sum(x, src2, dst2, zrow, zdeg, ones)

    blk = 256
    grid = (N_NODES + blk - 1) // blk
    s_term = pl.pallas_call(
        _tc_self_body,
        grid=(grid,),
        in_specs=[
            pl.BlockSpec((blk, D), lambda i: (i, 0)),
            pl.BlockSpec((D, D), lambda i: (0, 0)),
        ],
        out_specs=pl.BlockSpec((blk, D), lambda i: (i, 0)),
        out_shape=jax.ShapeDtypeStruct((N_NODES, D), jnp.float32),
    )(x, W_self)
    out = pl.pallas_call(
        _tc_combine_body,
        grid=(grid,),
        in_specs=[
            pl.BlockSpec((blk, D), lambda i: (i, 0)),
            pl.BlockSpec((blk, D), lambda i: (i, 0)),
            pl.BlockSpec((blk, 16), lambda i: (i, 0)),
            pl.BlockSpec((blk, 16), lambda i: (i, 0)),
            pl.BlockSpec((blk, D), lambda i: (i, 0)),
            pl.BlockSpec((D, D), lambda i: (0, 0)),
            pl.BlockSpec((1, D), lambda i: (0, 0)),
        ],
        out_specs=pl.BlockSpec((blk, D), lambda i: (i, 0)),
        out_shape=jax.ShapeDtypeStruct((N_NODES, D), jnp.float32),
    )(pooled[0], pooled[1], deg[0], deg[1], s_term, W_edge, b.reshape(1, D))
    return out


# speedup vs baseline: 1.0093x; 1.0093x over previous
import functools

import jax
import jax.numpy as jnp
from jax import lax
from jax.experimental import pallas as pl
from jax.experimental.pallas import tpu as pltpu
from jax.experimental.pallas import tpu_sc as plsc

N_NODES = 10000
N_EDGES = 320000
D = 128

NODES_PAD = 10112
TRASH_ROW = 10100
CHUNK = 128
NC = 2
NS = 16
NW = NC * NS
ROWS_PER_TILE = NODES_PAD // NS

CHUNKS_PER_TILE = 80
N_CHUNKS = NW * CHUNKS_PER_TILE
EDGES_PAD = N_CHUNKS * CHUNK
NBUF = 2


def _sc_segment_sum(x, src2, dst2, zrow, zdeg, ones):
    mesh = plsc.VectorSubcoreMesh(core_axis_name="c", subcore_axis_name="s")

    @functools.partial(
        pl.kernel,
        mesh=mesh,
        compiler_params=pltpu.CompilerParams(use_tc_tiling_on_sc=False),
        out_type=[
            jax.ShapeDtypeStruct((NC, NODES_PAD, D), jnp.float32),
            jax.ShapeDtypeStruct((NC, NODES_PAD, 16), jnp.float32),
        ],
        scratch_types=(
            [pltpu.VMEM((CHUNK,), jnp.int32) for _ in range(NBUF)]
            + [pltpu.VMEM((1, CHUNK), jnp.int32) for _ in range(NBUF)]
            + [pltpu.VMEM((CHUNK, D), jnp.float32) for _ in range(NBUF)]
            + [
                pltpu.VMEM((CHUNK, 16), jnp.float32),
                pltpu.VMEM_SHARED((NODES_PAD, D), jnp.float32),
                pltpu.VMEM_SHARED((NODES_PAD, 16), jnp.float32),
            ]
            + [pltpu.SemaphoreType.DMA for _ in range(3 * NBUF)]
        ),
    )
    def seg_kernel(x_hbm, src_hbm, dst_hbm, zrow_hbm, zdeg_hbm, ones_hbm,
                   pooled_hbm, deg_hbm, *refs):
        src_r = refs[0:NBUF]
        dst_r = refs[NBUF:2 * NBUF]
        buf_r = refs[2 * NBUF:3 * NBUF]
        ones_v = refs[3 * NBUF]
        accum_sh = refs[3 * NBUF + 1]
        deg_sh = refs[3 * NBUF + 2]
        gsem = refs[3 * NBUF + 3:4 * NBUF + 3]
        ssem = refs[4 * NBUF + 3:5 * NBUF + 3]
        dsem = refs[5 * NBUF + 3:6 * NBUF + 3]
        buf_a = buf_r[0]
        cid = lax.axis_index("c")
        sid = lax.axis_index("s")
        wid = sid * NC + cid

        r0 = sid * ROWS_PER_TILE
        pltpu.sync_copy(zrow_hbm, accum_sh.at[pl.ds(r0, ROWS_PER_TILE)])
        pltpu.sync_copy(zdeg_hbm, deg_sh.at[pl.ds(r0, ROWS_PER_TILE)])
        pltpu.sync_copy(ones_hbm, ones_v)
        e_base = wid * CHUNKS_PER_TILE * CHUNK
        plsc.subcore_barrier()

        def load_idx(k, i):
            e0 = e_base + k * CHUNK
            pltpu.sync_copy(src_hbm.at[pl.ds(e0, CHUNK)], src_r[i])
            pltpu.sync_copy(dst_hbm.at[pl.ds(e0, CHUNK)], dst_r[i].at[0])

        def gather(i):
            pltpu.make_async_copy(x_hbm.at[src_r[i]], buf_r[i], gsem[i]).start()

        def gwait(i):
            pltpu.make_async_copy(x_hbm.at[src_r[0]], buf_r[i], gsem[i]).wait()

        def scatter_start(i):
            pltpu.make_async_copy(
                buf_r[i], accum_sh.at[dst_r[i].at[0]], ssem[i]
            ).start(add=True)
            pltpu.make_async_copy(
                ones_v, deg_sh.at[dst_r[i].at[0]], dsem[i]
            ).start(add=True)

        def scatter_wait(i):
            pltpu.make_async_copy(
                buf_r[i], accum_sh.at[dst_r[i].at[0]], ssem[i]
            ).wait()
            pltpu.make_async_copy(
                ones_v, deg_sh.at[dst_r[i].at[0]], dsem[i]
            ).wait()

        for i in range(NBUF):
            load_idx(i, i)
            gather(i)

        def body(j, carry):
            k0 = NBUF * j
            for i in range(NBUF):
                gwait(i)
                scatter_start(i)

            @pl.when(j < CHUNKS_PER_TILE // NBUF - 1)
            def _():
                for i in range(NBUF):
                    scatter_wait(i)
                    load_idx(k0 + NBUF + i, i)
                    gather(i)

            @pl.when(j == CHUNKS_PER_TILE // NBUF - 1)
            def _():
                for i in range(NBUF):
                    scatter_wait(i)

            return carry

        lax.fori_loop(0, CHUNKS_PER_TILE // NBUF, body, 0)
        plsc.subcore_barrier()

        pltpu.sync_copy(accum_sh.at[pl.ds(r0, ROWS_PER_TILE)],
                        pooled_hbm.at[cid, pl.ds(r0, ROWS_PER_TILE)])
        pltpu.sync_copy(deg_sh.at[pl.ds(r0, ROWS_PER_TILE)],
                        deg_hbm.at[cid, pl.ds(r0, ROWS_PER_TILE)])

    return seg_kernel(x, src2, dst2, zrow, zdeg, ones)


def _tc_combine_body(p0, p1, d0, d1, x, we, ws, b, out):
    pooled = p0[...] + p1[...]
    e = jnp.dot(pooled, we[...], preferred_element_type=jnp.float32)
    s = jnp.dot(x[...], ws[...], preferred_element_type=jnp.float32)
    denom = d0[:, 0:1] + d1[:, 0:1] + 1.0
    out[...] = jnp.maximum((e + s) / denom + b[...], 0.0)


def kernel(x, edge_index, W_edge, W_self, b):
    src = edge_index[0]
    dst = edge_index[1]
    pad = EDGES_PAD - N_EDGES
    pad_idx = jnp.arange(pad, dtype=jnp.int32)
    src2 = jnp.concatenate([src, pad_idx % N_NODES])
    dst2 = jnp.concatenate([dst, N_NODES + pad_idx % (NODES_PAD - N_NODES)])
    zrow = jnp.zeros((ROWS_PER_TILE, D), jnp.float32)
    zdeg = jnp.zeros((ROWS_PER_TILE, 16), jnp.float32)
    ones = jnp.ones((CHUNK, 16), jnp.float32)

    pooled, deg = _sc_segment_sum(x, src2, dst2, zrow, zdeg, ones)

    blk = 256
    grid = (N_NODES + blk - 1) // blk
    out = pl.pallas_call(
        _tc_combine_body,
        grid=(grid,),
        in_specs=[
            pl.BlockSpec((blk, D), lambda i: (i, 0)),
            pl.BlockSpec((blk, D), lambda i: (i, 0)),
            pl.BlockSpec((blk, 16), lambda i: (i, 0)),
            pl.BlockSpec((blk, 16), lambda i: (i, 0)),
            pl.BlockSpec((blk, D), lambda i: (i, 0)),
            pl.BlockSpec((D, D), lambda i: (0, 0)),
            pl.BlockSpec((D, D), lambda i: (0, 0)),
            pl.BlockSpec((1, D), lambda i: (0, 0)),
        ],
        out_specs=pl.BlockSpec((blk, D), lambda i: (i, 0)),
        out_shape=jax.ShapeDtypeStruct((N_NODES, D), jnp.float32),
    )(pooled[0], pooled[1], deg[0], deg[1], x, W_edge, W_self, b.reshape(1, D))
    return out
